# baseline (speedup 1.0000x reference)
import jax
import jax.numpy as jnp
from jax.experimental import pallas as pl
from jax.experimental.pallas import tpu as pltpu

EMB = 256
CODES = 1024
PIX = 1024
BATCH = 16
COMMIT = 0.25


def _vq_kernel(x_ref, e_ref, out_ref, oh_ref, loss_ref, s_ref, et_ref, en_ref):
    n = pl.program_id(0)

    @pl.when(n == 0)
    def _():
        e = e_ref[...]
        et_ref[...] = e.T.astype(jnp.bfloat16)
        en_ref[...] = jnp.sum(e * e, axis=0, keepdims=True)
        loss_ref[...] = jnp.zeros_like(loss_ref)

    @pl.when(n < BATCH)
    def _():
        x = x_ref[...]
        s_ref[jax.lax.rem(n, 2)] = jax.lax.dot_general(
            x, e_ref[...], (((1,), (0,)), ((), ())),
            preferred_element_type=jnp.float32,
        )
        loss_ref[...] += jnp.sum(x * x)

    @pl.when(n > 0)
    def _():
        scores = s_ref[jax.lax.rem(n - 1, 2)]
        d = en_ref[...] - 2.0 * scores
        dmin = jnp.min(d, axis=1, keepdims=True)
        iota_c = jax.lax.broadcasted_iota(jnp.int32, (PIX, CODES), 1)
        idx = jnp.min(jnp.where(d == dmin, iota_c, CODES), axis=1,
                      keepdims=True)
        oh = jnp.where(iota_c == idx, 1.0, 0.0)
        oh_ref[...] = oh
        ohb = oh.astype(jnp.bfloat16)
        out_ref[...] = jax.lax.dot_general(
            ohb, et_ref[...], (((1,), (0,)), ((), ())),
            preferred_element_type=jnp.float32,
        )
        loss_ref[...] += jnp.sum(dmin)


def kernel(inputs, embed):
    flat = jnp.transpose(inputs, (0, 2, 3, 1)).reshape(BATCH * PIX, EMB)
    quant, onehot, loss_sum = pl.pallas_call(
        _vq_kernel,
        grid=(BATCH + 1,),
        in_specs=[
            pl.BlockSpec((PIX, EMB), lambda n: (jnp.minimum(n, BATCH - 1), 0)),
            pl.BlockSpec((EMB, CODES), lambda n: (0, 0)),
        ],
        out_specs=[
            pl.BlockSpec((PIX, EMB), lambda n: (jnp.maximum(n - 1, 0), 0)),
            pl.BlockSpec((PIX, CODES), lambda n: (jnp.maximum(n - 1, 0), 0)),
            pl.BlockSpec((1, 1), lambda n: (0, 0)),
        ],
        out_shape=[
            jax.ShapeDtypeStruct((BATCH * PIX, EMB), jnp.float32),
            jax.ShapeDtypeStruct((BATCH * PIX, CODES), jnp.float32),
            jax.ShapeDtypeStruct((1, 1), jnp.float32),
        ],
        scratch_shapes=[
            pltpu.VMEM((2, PIX, CODES), jnp.float32),
            pltpu.VMEM((CODES, EMB), jnp.bfloat16),
            pltpu.VMEM((1, CODES), jnp.float32),
        ],
        compiler_params=pltpu.CompilerParams(
            dimension_semantics=("arbitrary",),
        ),
    )(flat, embed)
    loss = loss_sum[0, 0] * ((1.0 + COMMIT) / (BATCH * PIX * EMB))
    out = jnp.transpose(quant.reshape(BATCH, 32, 32, EMB), (0, 3, 1, 2))
    return (out, loss, onehot)


# speedup vs baseline: 1.0869x; 1.0869x over previous
import jax
import jax.numpy as jnp
from jax.experimental import pallas as pl
from jax.experimental.pallas import tpu as pltpu

EMB = 256
CODES = 1024
PIX = 1024
BATCH = 16
ROWS = 2048
NSTEP = BATCH * PIX // ROWS
COMMIT = 0.25


def _vq_kernel(x_ref, e_ref, out_ref, oh_ref, loss_ref, s_ref, et_ref, en_ref):
    n = pl.program_id(0)

    @pl.when(n == 0)
    def _():
        e = e_ref[...]
        et_ref[...] = e.T
        en_ref[...] = jnp.sum(e * e, axis=0, keepdims=True)
        loss_ref[...] = jnp.zeros_like(loss_ref)

    @pl.when(n < NSTEP)
    def _():
        x = x_ref[...]
        s_ref[jax.lax.rem(n, 2)] = jax.lax.dot_general(
            x, e_ref[...], (((1,), (0,)), ((), ())),
            preferred_element_type=jnp.float32,
        )
        loss_ref[...] += jnp.sum(x * x)

    @pl.when(n > 0)
    def _():
        scores = s_ref[jax.lax.rem(n - 1, 2)]
        d = en_ref[...] - 2.0 * scores
        dmin = jnp.min(d, axis=1, keepdims=True)
        iota_c = jax.lax.broadcasted_iota(jnp.int32, (ROWS, CODES), 1)
        idx = jnp.min(jnp.where(d == dmin, iota_c, CODES), axis=1,
                      keepdims=True)
        oh = jnp.where(iota_c == idx, 1.0, 0.0)
        oh_ref[...] = oh
        out_ref[...] = jax.lax.dot_general(
            oh, et_ref[...], (((1,), (0,)), ((), ())),
            preferred_element_type=jnp.float32,
        )
        loss_ref[...] += jnp.sum(dmin)


def kernel(inputs, embed):
    flat = jnp.transpose(inputs, (0, 2, 3, 1)).reshape(BATCH * PIX, EMB)
    quant, onehot, loss_sum = pl.pallas_call(
        _vq_kernel,
        grid=(NSTEP + 1,),
        in_specs=[
            pl.BlockSpec((ROWS, EMB), lambda n: (jnp.minimum(n, NSTEP - 1), 0)),
            pl.BlockSpec((EMB, CODES), lambda n: (0, 0)),
        ],
        out_specs=[
            pl.BlockSpec((ROWS, EMB), lambda n: (jnp.maximum(n - 1, 0), 0)),
            pl.BlockSpec((ROWS, CODES), lambda n: (jnp.maximum(n - 1, 0), 0)),
            pl.BlockSpec((1, 1), lambda n: (0, 0)),
        ],
        out_shape=[
            jax.ShapeDtypeStruct((BATCH * PIX, EMB), jnp.float32),
            jax.ShapeDtypeStruct((BATCH * PIX, CODES), jnp.float32),
            jax.ShapeDtypeStruct((1, 1), jnp.float32),
        ],
        scratch_shapes=[
            pltpu.VMEM((2, ROWS, CODES), jnp.float32),
            pltpu.VMEM((CODES, EMB), jnp.float32),
            pltpu.VMEM((1, CODES), jnp.float32),
        ],
        compiler_params=pltpu.CompilerParams(
            dimension_semantics=("arbitrary",),
        ),
    )(flat, embed)
    loss = loss_sum[0, 0] * ((1.0 + COMMIT) / (BATCH * PIX * EMB))
    out = jnp.transpose(quant.reshape(BATCH, 32, 32, EMB), (0, 3, 1, 2))
    return (out, loss, onehot)
